# baseline (speedup 1.0000x reference)
import functools

import jax
import jax.numpy as jnp
from jax import lax
from jax.experimental import pallas as pl
from jax.experimental.pallas import tpu as pltpu, tpu_sc as plsc

N_ATOMS = 100000
EMB = 128
NROWS = 9

_info = plsc.get_sparse_core_info()
_NC, _NS = _info.num_cores, _info.num_subcores
_NW = _NC * _NS

_CH = 112
_G = _CH // 16
_QW = 3128
_Q_LAST = N_ATOMS - (_NW - 1) * _QW
_NFULL = _Q_LAST // _CH
_T_MAIN = _QW - _NFULL * _CH
_T_LAST = _Q_LAST - _NFULL * _CH
_IN_PAD = _QW * _NW

_mesh = plsc.VectorSubcoreMesh(core_axis_name="c", subcore_axis_name="s")


@functools.partial(
    pl.kernel,
    mesh=_mesh,
    compiler_params=pltpu.CompilerParams(needs_layout_passes=False),
    out_type=jax.ShapeDtypeStruct((N_ATOMS * EMB,), jnp.float32),
    scratch_types=[
        pltpu.VMEM((NROWS * EMB,), jnp.float32),
        pltpu.VMEM((_QW + 8,), jnp.int32),
        pltpu.VMEM((_CH * EMB,), jnp.float32),
        pltpu.VMEM((_CH * EMB,), jnp.float32),
        pltpu.SemaphoreType.DMA,
        pltpu.SemaphoreType.DMA,
    ],
)
def _emb_kernel(table_hbm, idx_hbm, out_hbm, tab_v, idx_v, buf_a, buf_b,
                sem_a, sem_b):
    wid = lax.axis_index("s") * _NC + lax.axis_index("c")
    base = wid * _QW
    pltpu.sync_copy(table_hbm, tab_v)
    pltpu.sync_copy(idx_hbm.at[pl.ds(base, _QW)], idx_v.at[pl.ds(0, _QW)])

    lane = lax.broadcasted_iota(jnp.int32, (16,), 0)
    bufs = (buf_a, buf_b)
    sems = (sem_a, sem_b)

    def copy_rows(cvec, rows, buf, mask=None):
        srcb = cvec * EMB
        dstb = rows * EMB
        for c in range(EMB):
            vals = plsc.load_gather(tab_v, [srcb + c], mask=mask)
            plsc.store_scatter(buf, [dstb + c], vals, mask=mask)

    def compute_chunk(koff, buf):
        def group(g, carry):
            cvec = idx_v[pl.ds(koff + g * 16, 16)]
            copy_rows(cvec, g * 16 + lane, buf)
            return carry
        lax.fori_loop(0, _G, group, 0)

    def drain(buf, sem):
        pltpu.make_async_copy(out_hbm.at[pl.ds(0, _CH * EMB)], buf, sem).wait()

    def pair(t, carry):
        for b in (0, 1):
            k = 2 * t + b

            @pl.when(t > 0)
            def _():
                drain(bufs[b], sems[b])

            compute_chunk(k * _CH, bufs[b])
            pltpu.async_copy(
                bufs[b],
                out_hbm.at[pl.ds((base + k * _CH) * EMB, _CH * EMB)],
                sems[b])
        return carry

    lax.fori_loop(0, (_NFULL - 1) // 2, pair, 0)

    drain(bufs[0], sems[0])
    compute_chunk((_NFULL - 1) * _CH, bufs[0])
    pltpu.async_copy(
        bufs[0],
        out_hbm.at[pl.ds((base + (_NFULL - 1) * _CH) * EMB, _CH * EMB)],
        sems[0])

    drain(bufs[1], sems[1])
    tail_off = _NFULL * _CH
    half = lane < 8

    @pl.when(wid < _NW - 1)
    def _():
        def group(g, carry):
            cvec = idx_v[pl.ds(tail_off + g * 16, 16)]
            copy_rows(cvec, g * 16 + lane, bufs[1])
            return carry
        lax.fori_loop(0, _T_MAIN // 16, group, 0)
        cvec = idx_v[pl.ds(tail_off + _T_MAIN - 8, 16)]
        copy_rows(cvec, (_T_MAIN - 8) + lane, bufs[1], mask=half)
        pltpu.async_copy(
            bufs[1].at[pl.ds(0, _T_MAIN * EMB)],
            out_hbm.at[pl.ds((base + tail_off) * EMB, _T_MAIN * EMB)],
            sems[1]).wait()

    @pl.when(wid == _NW - 1)
    def _():
        cvec = idx_v[pl.ds(tail_off, 16)]
        copy_rows(cvec, lane, bufs[1], mask=half)
        pltpu.async_copy(
            bufs[1].at[pl.ds(0, _T_LAST * EMB)],
            out_hbm.at[pl.ds((base + tail_off) * EMB, _T_LAST * EMB)],
            sems[1]).wait()

    drain(bufs[0], sems[0])


def kernel(C, table):
    idx = jnp.pad(C.astype(jnp.int32), (0, _IN_PAD - N_ATOMS))
    out = _emb_kernel(table.astype(jnp.float32).reshape(NROWS * EMB), idx)
    return out.reshape(N_ATOMS, EMB)


# speedup vs baseline: 4.2497x; 4.2497x over previous
import functools

import jax
import jax.numpy as jnp
from jax import lax
from jax.experimental import pallas as pl
from jax.experimental.pallas import tpu as pltpu, tpu_sc as plsc

N_ATOMS = 100000
EMB = 128
NROWS = 9

_info = plsc.get_sparse_core_info()
_NC, _NS = _info.num_cores, _info.num_subcores
_NW = _NC * _NS

_CH = 112
_G = _CH // 16
_QW = 3128
_Q_LAST = N_ATOMS - (_NW - 1) * _QW
_NFULL = _Q_LAST // _CH
_T_MAIN = _QW - _NFULL * _CH
_T_LAST = _Q_LAST - _NFULL * _CH
_IN_PAD = _QW * _NW

_mesh = plsc.VectorSubcoreMesh(core_axis_name="c", subcore_axis_name="s")


@functools.partial(
    pl.kernel,
    mesh=_mesh,
    compiler_params=pltpu.CompilerParams(needs_layout_passes=False),
    out_type=jax.ShapeDtypeStruct((N_ATOMS * EMB,), jnp.float32),
    scratch_types=[
        pltpu.VMEM((NROWS * EMB,), jnp.float32),
        pltpu.VMEM((_QW + 8,), jnp.int32),
        pltpu.VMEM((_CH * EMB,), jnp.float32),
        pltpu.VMEM((_CH * EMB,), jnp.float32),
        pltpu.SemaphoreType.DMA,
        pltpu.SemaphoreType.DMA,
    ],
)
def _emb_kernel(table_hbm, idx_hbm, out_hbm, tab_v, idx_v, buf_a, buf_b,
                sem_a, sem_b):
    wid = lax.axis_index("s") * _NC + lax.axis_index("c")
    base = wid * _QW
    pltpu.sync_copy(table_hbm, tab_v)
    pltpu.sync_copy(idx_hbm.at[pl.ds(base, _QW)], idx_v.at[pl.ds(0, _QW)])

    lane = lax.broadcasted_iota(jnp.int32, (16,), 0)
    bufs = (buf_a, buf_b)
    sems = (sem_a, sem_b)

    col16 = tuple(16 * j + lane for j in range(EMB // 16))

    def copy_rows(cvec, rows0, buf, atoms=16):
        srcb = cvec * EMB
        for a in range(atoms):
            sel = jnp.full((16,), a, jnp.int32)
            bca = srcb.at[sel].get(mode="promise_in_bounds")
            aoff = (rows0 + a) * EMB
            for j in range(EMB // 16):
                vals = plsc.load_gather(tab_v, [bca + col16[j]])
                buf[pl.ds(aoff + 16 * j, 16)] = vals

    def compute_chunk(koff, buf):
        def group(g, carry):
            cvec = idx_v[pl.ds(koff + g * 16, 16)]
            copy_rows(cvec, g * 16, buf)
            return carry
        lax.fori_loop(0, _G, group, 0)

    def drain(buf, sem):
        pltpu.make_async_copy(out_hbm.at[pl.ds(0, _CH * EMB)], buf, sem).wait()

    def pair(t, carry):
        for b in (0, 1):
            k = 2 * t + b

            @pl.when(t > 0)
            def _():
                drain(bufs[b], sems[b])

            compute_chunk(k * _CH, bufs[b])
            pltpu.async_copy(
                bufs[b],
                out_hbm.at[pl.ds((base + k * _CH) * EMB, _CH * EMB)],
                sems[b])
        return carry

    lax.fori_loop(0, (_NFULL - 1) // 2, pair, 0)

    drain(bufs[0], sems[0])
    compute_chunk((_NFULL - 1) * _CH, bufs[0])
    pltpu.async_copy(
        bufs[0],
        out_hbm.at[pl.ds((base + (_NFULL - 1) * _CH) * EMB, _CH * EMB)],
        sems[0])

    drain(bufs[1], sems[1])
    tail_off = _NFULL * _CH

    @pl.when(wid < _NW - 1)
    def _():
        def group(g, carry):
            cvec = idx_v[pl.ds(tail_off + g * 16, 16)]
            copy_rows(cvec, g * 16, bufs[1])
            return carry
        lax.fori_loop(0, _T_MAIN // 16, group, 0)
        cvec = idx_v[pl.ds(tail_off + _T_MAIN - 8, 16)]
        copy_rows(cvec, _T_MAIN - 8, bufs[1], atoms=8)
        pltpu.async_copy(
            bufs[1].at[pl.ds(0, _T_MAIN * EMB)],
            out_hbm.at[pl.ds((base + tail_off) * EMB, _T_MAIN * EMB)],
            sems[1]).wait()

    @pl.when(wid == _NW - 1)
    def _():
        cvec = idx_v[pl.ds(tail_off, 16)]
        copy_rows(cvec, 0, bufs[1], atoms=8)
        pltpu.async_copy(
            bufs[1].at[pl.ds(0, _T_LAST * EMB)],
            out_hbm.at[pl.ds((base + tail_off) * EMB, _T_LAST * EMB)],
            sems[1]).wait()

    drain(bufs[0], sems[0])


def kernel(C, table):
    idx = jnp.pad(C.astype(jnp.int32), (0, _IN_PAD - N_ATOMS))
    out = _emb_kernel(table.astype(jnp.float32).reshape(NROWS * EMB), idx)
    return out.reshape(N_ATOMS, EMB)


# speedup vs baseline: 11.1770x; 2.6301x over previous
import functools

import jax
import jax.numpy as jnp
from jax import lax
from jax.experimental import pallas as pl
from jax.experimental.pallas import tpu as pltpu, tpu_sc as plsc

N_ATOMS = 100000
EMB = 128
NROWS = 9

_info = plsc.get_sparse_core_info()
_NC, _NS = _info.num_cores, _info.num_subcores
_NW = _NC * _NS

_CH = 112
_QW = 3128
_Q_LAST = N_ATOMS - (_NW - 1) * _QW
_NFULL = _Q_LAST // _CH
_T_MAIN = _QW - _NFULL * _CH
_T_LAST = _Q_LAST - _NFULL * _CH
_IN_PAD = _QW * _NW
_NBUF = 4

_mesh = plsc.VectorSubcoreMesh(core_axis_name="c", subcore_axis_name="s")


@functools.partial(
    pl.kernel,
    mesh=_mesh,
    out_type=jax.ShapeDtypeStruct((N_ATOMS, EMB), jnp.float32),
    scratch_types=[
        pltpu.VMEM_SHARED((NROWS, EMB), jnp.float32),
        pltpu.VMEM((_QW,), jnp.int32),
        [pltpu.VMEM((_CH, EMB), jnp.float32) for _ in range(_NBUF)],
        [pltpu.SemaphoreType.DMA for _ in range(_NBUF)],
        [pltpu.SemaphoreType.DMA for _ in range(_NBUF)],
    ],
)
def _gather_kernel(table_hbm, idx_hbm, out_hbm, tab_sh, idx_v, bufs,
                   gsems, wsems):
    sid = lax.axis_index("s")
    wid = sid * _NC + lax.axis_index("c")
    base = wid * _QW

    @pl.when(sid == 0)
    def _():
        pltpu.sync_copy(table_hbm, tab_sh)

    pltpu.sync_copy(idx_hbm.at[pl.ds(base, _QW)], idx_v)
    plsc.subcore_barrier()

    def gather(k, b):
        return pltpu.async_copy(
            tab_sh.at[idx_v.at[pl.ds(k * _CH, _CH)]], bufs[b], gsems[b])

    gathers = [None] * _NFULL
    writes = [None] * _NFULL
    for j in range(_NBUF - 1):
        gathers[j] = gather(j, j)
    for k in range(_NFULL):
        b = k % _NBUF
        pre = k + _NBUF - 1
        if pre < _NFULL:
            bp = pre % _NBUF
            if k > 0:
                writes[k - 1].wait()
            gathers[pre] = gather(pre, bp)
        gathers[k].wait()
        writes[k] = pltpu.async_copy(
            bufs[b], out_hbm.at[pl.ds(base + k * _CH, _CH)], wsems[b])
    for k in range(_NFULL - _NBUF, _NFULL):
        writes[k].wait()

    tail_off = _NFULL * _CH

    @pl.when(wid < _NW - 1)
    def _():
        tb = bufs[0].at[pl.ds(0, _T_MAIN)]
        pltpu.async_copy(
            tab_sh.at[idx_v.at[pl.ds(tail_off, _T_MAIN)]],
            tb, gsems[0]).wait()
        pltpu.sync_copy(tb, out_hbm.at[pl.ds(base + tail_off, _T_MAIN)])

    @pl.when(wid == _NW - 1)
    def _():
        tb = bufs[1].at[pl.ds(0, _T_LAST)]
        pltpu.async_copy(
            tab_sh.at[idx_v.at[pl.ds(tail_off, _T_LAST)]],
            tb, gsems[1]).wait()
        pltpu.sync_copy(tb, out_hbm.at[pl.ds(base + tail_off, _T_LAST)])


def kernel(C, table):
    idx = jnp.pad(C.astype(jnp.int32), (0, _IN_PAD - N_ATOMS))
    return _gather_kernel(table.astype(jnp.float32), idx)
